# baseline (speedup 1.0000x reference)
import functools

import jax
import jax.numpy as jnp
from jax import lax
from jax.experimental import pallas as pl
from jax.experimental.pallas import tpu as pltpu
from jax.experimental.pallas import tpu_sc as plsc

MAP_SIZE = 50
NCELL = MAP_SIZE * MAP_SIZE
NCELL_PAD = 2560
BRAIN_DIM = 512
N_PLACE = 100
B = 16384
H = 256

NC = 2
NS = 16
L = 16
NW = NC * NS
BPW = B // NW
CH = 8
CB = BPW // CH


@functools.cache
def _sc_part_fn():
    mesh = plsc.VectorSubcoreMesh(core_axis_name="c", subcore_axis_name="s")

    @functools.partial(
        pl.kernel,
        mesh=mesh,
        out_type=[
            jax.ShapeDtypeStruct((1, B), jnp.float32),
            jax.ShapeDtypeStruct((B, BRAIN_DIM), jnp.float32),
            jax.ShapeDtypeStruct((NC * NCELL_PAD,), jnp.float32),
        ],
        scratch_types=[
            pltpu.VMEM((BPW,), jnp.float32),
            pltpu.VMEM((BPW,), jnp.float32),
            pltpu.VMEM((NCELL_PAD,), jnp.float32),
            pltpu.VMEM((CH, CB), jnp.int32),
            pltpu.VMEM((BPW,), jnp.float32),
            pltpu.VMEM((CB,), jnp.float32),
            pltpu.VMEM((CB, BRAIN_DIM), jnp.float32),
            pltpu.VMEM((CB, BRAIN_DIM), jnp.float32),
            pltpu.VMEM((NCELL_PAD,), jnp.float32),
            pltpu.VMEM_SHARED((NCELL_PAD,), jnp.float32),
            pltpu.SemaphoreType.DMA,
            pltpu.SemaphoreType.DMA,
            pltpu.SemaphoreType.DMA,
            pltpu.SemaphoreType.DMA,
            pltpu.SemaphoreType.DMA,
        ],
        compiler_params=pltpu.CompilerParams(needs_layout_passes=False),
    )
    def _sc_part(pos_hbm, vc_hbm, sm_hbm, nov_hbm, map_hbm, part_hbm,
                 posx_v, posy_v, vc_v, idx_v, nov_v, ones_v, rows_a, rows_b,
                 zer_v, hist_sh, sem_in, sem_nov, sem_sa, sem_g, sem_w):
        cid = lax.axis_index("c")
        sid = lax.axis_index("s")
        wid = sid * NC + cid
        base = wid * BPW

        cpx = pltpu.async_copy(pos_hbm.at[pl.ds(base, BPW)], posx_v, sem_in)
        cpy = pltpu.async_copy(pos_hbm.at[pl.ds(B + base, BPW)], posy_v,
                               sem_in)
        cvc = pltpu.async_copy(vc_hbm, vc_v, sem_in)

        def obody(i, _):
            ones_v[pl.ds(i * L, L)] = jnp.ones((L,), jnp.float32)
            return 0
        lax.fori_loop(0, CB // L, obody, 0)

        @pl.when(sid == 0)
        def _init_hist():
            def zbody(i, _):
                zer_v[pl.ds(i * L, L)] = jnp.zeros((L,), jnp.float32)
                return 0
            lax.fori_loop(0, NCELL_PAD // L, zbody, 0)
            pltpu.sync_copy(zer_v, hist_sh)

        cpx.wait()
        cpy.wait()
        cvc.wait()

        for c in range(CH):
            def ibody(j, _, c=c):
                i = c * (CB // L) + j
                xs = posx_v[pl.ds(i * L, L)]
                ys = posy_v[pl.ds(i * L, L)]
                gx = jnp.clip((xs * MAP_SIZE).astype(jnp.int32),
                              0, MAP_SIZE - 1)
                gy = jnp.clip((ys * MAP_SIZE).astype(jnp.int32),
                              0, MAP_SIZE - 1)
                gi = gx * MAP_SIZE + gy
                idx_v[c, pl.ds(j * L, L)] = gi
                counts = plsc.load_gather(vc_v, [gi])
                nov_v[pl.ds(i * L, L)] = jnp.exp(counts * (-0.1))
                return 0
            lax.fori_loop(0, CB // L, ibody, 0)

        nv = pltpu.async_copy(nov_v, nov_hbm.at[0, pl.ds(base, BPW)], sem_nov)

        plsc.subcore_barrier()
        sa = [pltpu.async_copy(ones_v, hist_sh.at[idx_v.at[c]], sem_sa,
                               add=True)
              for c in range(CH)]

        bufs = (rows_a, rows_b)
        gd = [None] * CH
        wb = [None] * CH
        gd[0] = pltpu.async_copy(sm_hbm.at[idx_v.at[0]], bufs[0], sem_g)
        for c in range(CH):
            gd[c].wait()
            if c + 1 < CH:
                if c >= 1:
                    wb[c - 1].wait()
                gd[c + 1] = pltpu.async_copy(sm_hbm.at[idx_v.at[c + 1]],
                                             bufs[(c + 1) % 2], sem_g)
            wb[c] = pltpu.async_copy(bufs[c % 2],
                                     map_hbm.at[pl.ds(base + c * CB, CB)],
                                     sem_w)
        wb[CH - 2].wait()
        wb[CH - 1].wait()
        for d in sa:
            d.wait()
        nv.wait()

        plsc.subcore_barrier()
        pltpu.sync_copy(hist_sh.at[pl.ds(sid * 160, 160)],
                        zer_v.at[pl.ds(0, 160)])
        pltpu.sync_copy(zer_v.at[pl.ds(0, 160)],
                        part_hbm.at[pl.ds(cid * NCELL_PAD + sid * 160, 160)])

    return _sc_part


BLK = 2048
_GRID = B // BLK


def _tc_body(bs_ref, pos_ref, cx_ref, cy_ref, iv_ref, w1_ref, b1_ref,
             w2_ref, b2_ref, wdt_ref, bd_ref, wst_ref, bsc_ref, wgt_ref,
             bg_ref, dirp_ref, spd_ref, gate_ref, pc_ref):
    bf = jnp.bfloat16
    px = pos_ref[0:1, :]
    py = pos_ref[1:2, :]
    d2 = (cx_ref[:] - px) ** 2 + (cy_ref[:] - py) ** 2
    pc_t = jnp.exp(-d2 * iv_ref[:])
    pc_ref[:] = pc_t
    f = jnp.maximum(
        jnp.dot(bs_ref[:].astype(bf), w1_ref[0:BRAIN_DIM, :].astype(bf),
                preferred_element_type=jnp.float32)
        + lax.dot_general(pc_t.astype(bf),
                          w1_ref[BRAIN_DIM:BRAIN_DIM + N_PLACE, :].astype(bf),
                          (((0,), (0,)), ((), ())),
                          preferred_element_type=jnp.float32)
        + b1_ref[:], 0.0)
    fb = f.astype(bf)
    f2 = jnp.maximum(
        jnp.dot(fb, w2_ref[:].astype(bf), preferred_element_type=jnp.float32)
        + b2_ref[:], 0.0)
    f2b = f2.astype(bf)
    logits = lax.dot_general(wdt_ref[:].astype(bf), f2b,
                             (((1,), (1,)), ((), ())),
                             preferred_element_type=jnp.float32) + bd_ref[:]
    m = jnp.max(logits, axis=0, keepdims=True)
    e = jnp.exp(logits - m)
    dirp_ref[:] = e / jnp.sum(e, axis=0, keepdims=True)
    s = lax.dot_general(wst_ref[:].astype(bf), f2b,
                        (((1,), (1,)), ((), ())),
                        preferred_element_type=jnp.float32) + bsc_ref[:]
    spd_ref[:] = 1.0 / (1.0 + jnp.exp(-s))
    g = lax.dot_general(wgt_ref[:].astype(bf), f2b,
                        (((1,), (1,)), ((), ())),
                        preferred_element_type=jnp.float32) + bg_ref[:]
    gate_ref[:] = 1.0 / (1.0 + jnp.exp(-g))


def _tc_call(interpret=False):
    def row_blk(shape):
        return pl.BlockSpec(shape, lambda i: (i, 0))

    def col_blk(shape):
        return pl.BlockSpec(shape, lambda i: (0, i))

    def rep_blk(shape):
        return pl.BlockSpec(shape, lambda i: (0, 0))

    return pl.pallas_call(
        _tc_body,
        grid=(_GRID,),
        in_specs=[
            row_blk((BLK, BRAIN_DIM)),
            col_blk((2, BLK)),
            rep_blk((N_PLACE, 1)),
            rep_blk((N_PLACE, 1)),
            rep_blk((N_PLACE, 1)),
            rep_blk((BRAIN_DIM + N_PLACE, H)),
            rep_blk((1, H)),
            rep_blk((H, H)),
            rep_blk((1, H)),
            rep_blk((9, H)),
            rep_blk((9, 1)),
            rep_blk((1, H)),
            rep_blk((1, 1)),
            rep_blk((1, H)),
            rep_blk((1, 1)),
        ],
        out_specs=[
            col_blk((9, BLK)),
            col_blk((1, BLK)),
            col_blk((1, BLK)),
            col_blk((N_PLACE, BLK)),
        ],
        out_shape=[
            jax.ShapeDtypeStruct((9, B), jnp.float32),
            jax.ShapeDtypeStruct((1, B), jnp.float32),
            jax.ShapeDtypeStruct((1, B), jnp.float32),
            jax.ShapeDtypeStruct((N_PLACE, B), jnp.float32),
        ],
        compiler_params=pltpu.CompilerParams(
            dimension_semantics=("parallel",)),
        interpret=interpret,
    )


def kernel(brain_state, position, place_cell_centers, place_cell_widths,
           visit_count, spatial_map, W1, b1, W2, b2, Wd, bd, Ws, bs, Wg, bg):
    pos_t = position.T
    posf = pos_t.reshape(-1)
    vcp = jnp.pad(visit_count.reshape(-1), (0, NCELL_PAD - NCELL))
    smf = spatial_map.reshape(NCELL, BRAIN_DIM)
    nov_t, mapf, parts = _sc_part_fn()(posf, vcp, smf)

    w = place_cell_widths.reshape(N_PLACE, 1)
    dirp_t, spd_t, gate_t, pc_t = _tc_call()(
        brain_state, pos_t,
        place_cell_centers[:, 0:1],
        place_cell_centers[:, 1:2],
        1.0 / (2.0 * w * w),
        W1, b1.reshape(1, H),
        W2, b2.reshape(1, H),
        Wd.T, bd.reshape(9, 1),
        Ws.T, bs.reshape(1, 1),
        Wg.T, bg.reshape(1, 1),
    )

    parts = parts.reshape(NC, NCELL_PAD)
    nvc = visit_count + (parts[0, :NCELL] + parts[1, :NCELL]).reshape(
        MAP_SIZE, MAP_SIZE)
    return (dirp_t.T, spd_t.T, gate_t.T, nov_t.T, pc_t.T, mapf, nvc)


# speedup vs baseline: 1.0195x; 1.0195x over previous
import functools

import jax
import jax.numpy as jnp
from jax import lax
from jax.experimental import pallas as pl
from jax.experimental.pallas import tpu as pltpu
from jax.experimental.pallas import tpu_sc as plsc

MAP_SIZE = 50
NCELL = MAP_SIZE * MAP_SIZE
NCELL_PAD = 2560
BRAIN_DIM = 512
N_PLACE = 100
B = 16384
H = 256

NC = 2
NS = 16
L = 16
NW = NC * NS
BPW = B // NW
CH = 8
CB = BPW // CH


@functools.cache
def _sc_part_fn():
    mesh = plsc.VectorSubcoreMesh(core_axis_name="c", subcore_axis_name="s")

    @functools.partial(
        pl.kernel,
        mesh=mesh,
        out_type=[
            jax.ShapeDtypeStruct((1, B), jnp.float32),
            jax.ShapeDtypeStruct((B, BRAIN_DIM), jnp.float32),
            jax.ShapeDtypeStruct((NC * NCELL_PAD,), jnp.float32),
        ],
        scratch_types=[
            pltpu.VMEM((BPW,), jnp.float32),
            pltpu.VMEM((BPW,), jnp.float32),
            pltpu.VMEM((NCELL_PAD,), jnp.float32),
            pltpu.VMEM((CH, CB), jnp.int32),
            pltpu.VMEM((BPW,), jnp.float32),
            pltpu.VMEM((CB,), jnp.float32),
            pltpu.VMEM((CB, BRAIN_DIM), jnp.float32),
            pltpu.VMEM((CB, BRAIN_DIM), jnp.float32),
            pltpu.VMEM((NCELL_PAD,), jnp.float32),
            pltpu.VMEM_SHARED((NCELL_PAD,), jnp.float32),
            pltpu.SemaphoreType.DMA,
            pltpu.SemaphoreType.DMA,
            pltpu.SemaphoreType.DMA,
            pltpu.SemaphoreType.DMA,
            pltpu.SemaphoreType.DMA,
        ],
        compiler_params=pltpu.CompilerParams(needs_layout_passes=False),
    )
    def _sc_part(pos_hbm, vc_hbm, sm_hbm, nov_hbm, map_hbm, part_hbm,
                 posx_v, posy_v, vc_v, idx_v, nov_v, ones_v, rows_a, rows_b,
                 zer_v, hist_sh, sem_in, sem_nov, sem_sa, sem_g, sem_w):
        cid = lax.axis_index("c")
        sid = lax.axis_index("s")
        wid = sid * NC + cid
        base = wid * BPW

        cpx = pltpu.async_copy(pos_hbm.at[pl.ds(base, BPW)], posx_v, sem_in)
        cpy = pltpu.async_copy(pos_hbm.at[pl.ds(B + base, BPW)], posy_v,
                               sem_in)
        cvc = pltpu.async_copy(vc_hbm, vc_v, sem_in)

        def obody(i, _):
            ones_v[pl.ds(i * L, L)] = jnp.ones((L,), jnp.float32)
            return 0
        lax.fori_loop(0, CB // L, obody, 0)

        @pl.when(sid == 0)
        def _init_hist():
            def zbody(i, _):
                zer_v[pl.ds(i * L, L)] = jnp.zeros((L,), jnp.float32)
                return 0
            lax.fori_loop(0, NCELL_PAD // L, zbody, 0)
            pltpu.sync_copy(zer_v, hist_sh)

        cpx.wait()
        cpy.wait()
        cvc.wait()

        for c in range(CH):
            def ibody(j, _, c=c):
                i = c * (CB // L) + j
                xs = posx_v[pl.ds(i * L, L)]
                ys = posy_v[pl.ds(i * L, L)]
                gx = jnp.clip((xs * MAP_SIZE).astype(jnp.int32),
                              0, MAP_SIZE - 1)
                gy = jnp.clip((ys * MAP_SIZE).astype(jnp.int32),
                              0, MAP_SIZE - 1)
                gi = gx * MAP_SIZE + gy
                idx_v[c, pl.ds(j * L, L)] = gi
                counts = plsc.load_gather(vc_v, [gi])
                nov_v[pl.ds(i * L, L)] = jnp.exp(counts * (-0.1))
                return 0
            lax.fori_loop(0, CB // L, ibody, 0)

        nv = pltpu.async_copy(nov_v, nov_hbm.at[0, pl.ds(base, BPW)], sem_nov)

        plsc.subcore_barrier()
        sa = [pltpu.async_copy(ones_v, hist_sh.at[idx_v.at[c]], sem_sa,
                               add=True)
              for c in range(CH)]

        bufs = (rows_a, rows_b)
        gd = [None] * CH
        wb = [None] * CH
        gd[0] = pltpu.async_copy(sm_hbm.at[idx_v.at[0]], bufs[0], sem_g)
        for c in range(CH):
            gd[c].wait()
            if c + 1 < CH:
                if c >= 1:
                    wb[c - 1].wait()
                gd[c + 1] = pltpu.async_copy(sm_hbm.at[idx_v.at[c + 1]],
                                             bufs[(c + 1) % 2], sem_g)
            wb[c] = pltpu.async_copy(bufs[c % 2],
                                     map_hbm.at[pl.ds(base + c * CB, CB)],
                                     sem_w)
        wb[CH - 2].wait()
        wb[CH - 1].wait()
        for d in sa:
            d.wait()
        nv.wait()

        plsc.subcore_barrier()
        pltpu.sync_copy(hist_sh.at[pl.ds(sid * 160, 160)],
                        zer_v.at[pl.ds(0, 160)])
        pltpu.sync_copy(zer_v.at[pl.ds(0, 160)],
                        part_hbm.at[pl.ds(cid * NCELL_PAD + sid * 160, 160)])

    return _sc_part


BLK = 4096
_GRID = B // BLK


def _tc_body(bs_ref, pos_ref, cx_ref, cy_ref, iv_ref, w1_ref, b1_ref,
             w2_ref, b2_ref, wdt_ref, bd_ref, wst_ref, bsc_ref, wgt_ref,
             bg_ref, dirp_ref, spd_ref, gate_ref, pc_ref):
    bf = jnp.bfloat16
    px = pos_ref[0:1, :]
    py = pos_ref[1:2, :]
    d2 = (cx_ref[:] - px) ** 2 + (cy_ref[:] - py) ** 2
    pc_t = jnp.exp(-d2 * iv_ref[:])
    pc_ref[:] = pc_t
    f = jnp.maximum(
        jnp.dot(bs_ref[:].astype(bf), w1_ref[0:BRAIN_DIM, :].astype(bf),
                preferred_element_type=jnp.float32)
        + lax.dot_general(pc_t.astype(bf),
                          w1_ref[BRAIN_DIM:BRAIN_DIM + N_PLACE, :].astype(bf),
                          (((0,), (0,)), ((), ())),
                          preferred_element_type=jnp.float32)
        + b1_ref[:], 0.0)
    fb = f.astype(bf)
    f2 = jnp.maximum(
        jnp.dot(fb, w2_ref[:].astype(bf), preferred_element_type=jnp.float32)
        + b2_ref[:], 0.0)
    f2b = f2.astype(bf)
    logits = lax.dot_general(wdt_ref[:].astype(bf), f2b,
                             (((1,), (1,)), ((), ())),
                             preferred_element_type=jnp.float32) + bd_ref[:]
    m = jnp.max(logits, axis=0, keepdims=True)
    e = jnp.exp(logits - m)
    dirp_ref[:] = e / jnp.sum(e, axis=0, keepdims=True)
    s = lax.dot_general(wst_ref[:].astype(bf), f2b,
                        (((1,), (1,)), ((), ())),
                        preferred_element_type=jnp.float32) + bsc_ref[:]
    spd_ref[:] = 1.0 / (1.0 + jnp.exp(-s))
    g = lax.dot_general(wgt_ref[:].astype(bf), f2b,
                        (((1,), (1,)), ((), ())),
                        preferred_element_type=jnp.float32) + bg_ref[:]
    gate_ref[:] = 1.0 / (1.0 + jnp.exp(-g))


def _tc_call(interpret=False):
    def row_blk(shape):
        return pl.BlockSpec(shape, lambda i: (i, 0))

    def col_blk(shape):
        return pl.BlockSpec(shape, lambda i: (0, i))

    def rep_blk(shape):
        return pl.BlockSpec(shape, lambda i: (0, 0))

    return pl.pallas_call(
        _tc_body,
        grid=(_GRID,),
        in_specs=[
            row_blk((BLK, BRAIN_DIM)),
            col_blk((2, BLK)),
            rep_blk((N_PLACE, 1)),
            rep_blk((N_PLACE, 1)),
            rep_blk((N_PLACE, 1)),
            rep_blk((BRAIN_DIM + N_PLACE, H)),
            rep_blk((1, H)),
            rep_blk((H, H)),
            rep_blk((1, H)),
            rep_blk((9, H)),
            rep_blk((9, 1)),
            rep_blk((1, H)),
            rep_blk((1, 1)),
            rep_blk((1, H)),
            rep_blk((1, 1)),
        ],
        out_specs=[
            col_blk((9, BLK)),
            col_blk((1, BLK)),
            col_blk((1, BLK)),
            col_blk((N_PLACE, BLK)),
        ],
        out_shape=[
            jax.ShapeDtypeStruct((9, B), jnp.float32),
            jax.ShapeDtypeStruct((1, B), jnp.float32),
            jax.ShapeDtypeStruct((1, B), jnp.float32),
            jax.ShapeDtypeStruct((N_PLACE, B), jnp.float32),
        ],
        compiler_params=pltpu.CompilerParams(
            dimension_semantics=("parallel",)),
        interpret=interpret,
    )


def kernel(brain_state, position, place_cell_centers, place_cell_widths,
           visit_count, spatial_map, W1, b1, W2, b2, Wd, bd, Ws, bs, Wg, bg):
    pos_t = position.T
    posf = pos_t.reshape(-1)
    vcp = jnp.pad(visit_count.reshape(-1), (0, NCELL_PAD - NCELL))
    smf = spatial_map.reshape(NCELL, BRAIN_DIM)
    nov_t, mapf, parts = _sc_part_fn()(posf, vcp, smf)

    w = place_cell_widths.reshape(N_PLACE, 1)
    dirp_t, spd_t, gate_t, pc_t = _tc_call()(
        brain_state, pos_t,
        place_cell_centers[:, 0:1],
        place_cell_centers[:, 1:2],
        1.0 / (2.0 * w * w),
        W1, b1.reshape(1, H),
        W2, b2.reshape(1, H),
        Wd.T, bd.reshape(9, 1),
        Ws.T, bs.reshape(1, 1),
        Wg.T, bg.reshape(1, 1),
    )

    parts = parts.reshape(NC, NCELL_PAD)
    nvc = visit_count + (parts[0, :NCELL] + parts[1, :NCELL]).reshape(
        MAP_SIZE, MAP_SIZE)
    return (dirp_t.T, spd_t.T, gate_t.T, nov_t.T, pc_t.T, mapf, nvc)


# speedup vs baseline: 1.0517x; 1.0315x over previous
import functools

import jax
import jax.numpy as jnp
from jax import lax
from jax.experimental import pallas as pl
from jax.experimental.pallas import tpu as pltpu
from jax.experimental.pallas import tpu_sc as plsc

MAP_SIZE = 50
NCELL = MAP_SIZE * MAP_SIZE
NCELL_PAD = 2560
BRAIN_DIM = 512
N_PLACE = 100
B = 16384
H = 256

NC = 2
NS = 16
L = 16
NW = NC * NS
BPW = B // NW
CH = 8
CB = BPW // CH


@functools.cache
def _sc_part_fn():
    mesh = plsc.VectorSubcoreMesh(core_axis_name="c", subcore_axis_name="s")

    @functools.partial(
        pl.kernel,
        mesh=mesh,
        out_type=[
            jax.ShapeDtypeStruct((1, B), jnp.float32),
            jax.ShapeDtypeStruct((B, BRAIN_DIM), jnp.float32),
            jax.ShapeDtypeStruct((NC * NCELL_PAD,), jnp.float32),
        ],
        scratch_types=[
            pltpu.VMEM((BPW,), jnp.float32),
            pltpu.VMEM((BPW,), jnp.float32),
            pltpu.VMEM((NCELL_PAD,), jnp.float32),
            pltpu.VMEM((CH, CB), jnp.int32),
            pltpu.VMEM((BPW,), jnp.float32),
            pltpu.VMEM((CB,), jnp.float32),
            pltpu.VMEM((CB, BRAIN_DIM), jnp.float32),
            pltpu.VMEM((CB, BRAIN_DIM), jnp.float32),
            pltpu.VMEM((NCELL_PAD,), jnp.float32),
            pltpu.VMEM_SHARED((NCELL_PAD,), jnp.float32),
            pltpu.SemaphoreType.DMA,
            pltpu.SemaphoreType.DMA,
            pltpu.SemaphoreType.DMA,
            pltpu.SemaphoreType.DMA,
            pltpu.SemaphoreType.DMA,
        ],
        compiler_params=pltpu.CompilerParams(needs_layout_passes=False),
    )
    def _sc_part(pos_hbm, vc_hbm, sm_hbm, nov_hbm, map_hbm, part_hbm,
                 posx_v, posy_v, vc_v, idx_v, nov_v, ones_v, rows_a, rows_b,
                 zer_v, hist_sh, sem_in, sem_nov, sem_sa, sem_g, sem_w):
        cid = lax.axis_index("c")
        sid = lax.axis_index("s")
        wid = sid * NC + cid
        base = wid * BPW

        cpx = pltpu.async_copy(pos_hbm.at[pl.ds(base, BPW)], posx_v, sem_in)
        cpy = pltpu.async_copy(pos_hbm.at[pl.ds(B + base, BPW)], posy_v,
                               sem_in)
        cvc = pltpu.async_copy(vc_hbm, vc_v, sem_in)

        def obody(i, _):
            ones_v[pl.ds(i * L, L)] = jnp.ones((L,), jnp.float32)
            return 0
        lax.fori_loop(0, CB // L, obody, 0)

        @pl.when(sid == 0)
        def _init_hist():
            def zbody(i, _):
                zer_v[pl.ds(i * L, L)] = jnp.zeros((L,), jnp.float32)
                return 0
            lax.fori_loop(0, NCELL_PAD // L, zbody, 0)
            pltpu.sync_copy(zer_v, hist_sh)

        cpx.wait()
        cpy.wait()
        cvc.wait()

        for c in range(CH):
            def ibody(j, _, c=c):
                i = c * (CB // L) + j
                xs = posx_v[pl.ds(i * L, L)]
                ys = posy_v[pl.ds(i * L, L)]
                gx = jnp.clip((xs * MAP_SIZE).astype(jnp.int32),
                              0, MAP_SIZE - 1)
                gy = jnp.clip((ys * MAP_SIZE).astype(jnp.int32),
                              0, MAP_SIZE - 1)
                gi = gx * MAP_SIZE + gy
                idx_v[c, pl.ds(j * L, L)] = gi
                counts = plsc.load_gather(vc_v, [gi])
                nov_v[pl.ds(i * L, L)] = jnp.exp(counts * (-0.1))
                return 0
            lax.fori_loop(0, CB // L, ibody, 0)

        nv = pltpu.async_copy(nov_v, nov_hbm.at[0, pl.ds(base, BPW)], sem_nov)

        plsc.subcore_barrier()
        sa = [pltpu.async_copy(ones_v, hist_sh.at[idx_v.at[c]], sem_sa,
                               add=True)
              for c in range(CH)]

        bufs = (rows_a, rows_b)
        gd = [None] * CH
        wb = [None] * CH
        gd[0] = pltpu.async_copy(sm_hbm.at[idx_v.at[0]], bufs[0], sem_g)
        for c in range(CH):
            gd[c].wait()
            if c + 1 < CH:
                if c >= 1:
                    wb[c - 1].wait()
                gd[c + 1] = pltpu.async_copy(sm_hbm.at[idx_v.at[c + 1]],
                                             bufs[(c + 1) % 2], sem_g)
            wb[c] = pltpu.async_copy(bufs[c % 2],
                                     map_hbm.at[pl.ds(base + c * CB, CB)],
                                     sem_w)
        wb[CH - 2].wait()
        wb[CH - 1].wait()
        for d in sa:
            d.wait()
        nv.wait()

        plsc.subcore_barrier()
        pltpu.sync_copy(hist_sh.at[pl.ds(sid * 160, 160)],
                        zer_v.at[pl.ds(0, 160)])
        pltpu.sync_copy(zer_v.at[pl.ds(0, 160)],
                        part_hbm.at[pl.ds(cid * NCELL_PAD + sid * 160, 160)])

    return _sc_part


BLK = 4096
_GRID = B // BLK


def _tc_body(bs_ref, pos_ref, ct_ref, w_ref, w1_ref, b1_ref,
             w2_ref, b2_ref, wdt_ref, bd_ref, wst_ref, bsc_ref, wgt_ref,
             bg_ref, dirp_ref, spd_ref, gate_ref, pc_ref):
    bf = jnp.bfloat16
    px = pos_ref[0:1, :]
    py = pos_ref[1:2, :]
    cx = ct_ref[0:1, :]
    cy = ct_ref[1:2, :]
    w = w_ref[:]
    iv = 0.5 / (w * w)
    a_mat = jnp.concatenate(
        [(cx * cx + cy * cy) * iv, -2.0 * cx * iv, -2.0 * cy * iv, iv, iv],
        axis=0)
    b_mat = jnp.concatenate(
        [jnp.ones_like(px), px, py, px * px, py * py], axis=0)
    d2iv = lax.dot_general(a_mat, b_mat, (((0,), (0,)), ((), ())),
                           preferred_element_type=jnp.float32)
    pc_t = jnp.exp(-d2iv)
    pc_ref[:] = pc_t
    f = jnp.maximum(
        jnp.dot(bs_ref[:].astype(bf), w1_ref[0:BRAIN_DIM, :].astype(bf),
                preferred_element_type=jnp.float32)
        + lax.dot_general(pc_t.astype(bf),
                          w1_ref[BRAIN_DIM:BRAIN_DIM + N_PLACE, :].astype(bf),
                          (((0,), (0,)), ((), ())),
                          preferred_element_type=jnp.float32)
        + b1_ref[:], 0.0)
    fb = f.astype(bf)
    f2 = jnp.maximum(
        jnp.dot(fb, w2_ref[:].astype(bf), preferred_element_type=jnp.float32)
        + b2_ref[:], 0.0)
    f2b = f2.astype(bf)
    logits = lax.dot_general(wdt_ref[:].astype(bf), f2b,
                             (((1,), (1,)), ((), ())),
                             preferred_element_type=jnp.float32) + bd_ref[:]
    m = jnp.max(logits, axis=0, keepdims=True)
    e = jnp.exp(logits - m)
    dirp_ref[:] = e / jnp.sum(e, axis=0, keepdims=True)
    s = lax.dot_general(wst_ref[:].astype(bf), f2b,
                        (((1,), (1,)), ((), ())),
                        preferred_element_type=jnp.float32) + bsc_ref[:]
    spd_ref[:] = 1.0 / (1.0 + jnp.exp(-s))
    g = lax.dot_general(wgt_ref[:].astype(bf), f2b,
                        (((1,), (1,)), ((), ())),
                        preferred_element_type=jnp.float32) + bg_ref[:]
    gate_ref[:] = 1.0 / (1.0 + jnp.exp(-g))


def _tc_call(interpret=False):
    def row_blk(shape):
        return pl.BlockSpec(shape, lambda i: (i, 0))

    def col_blk(shape):
        return pl.BlockSpec(shape, lambda i: (0, i))

    def rep_blk(shape):
        return pl.BlockSpec(shape, lambda i: (0, 0))

    return pl.pallas_call(
        _tc_body,
        grid=(_GRID,),
        in_specs=[
            row_blk((BLK, BRAIN_DIM)),
            col_blk((2, BLK)),
            rep_blk((2, N_PLACE)),
            rep_blk((1, N_PLACE)),
            rep_blk((BRAIN_DIM + N_PLACE, H)),
            rep_blk((1, H)),
            rep_blk((H, H)),
            rep_blk((1, H)),
            rep_blk((9, H)),
            rep_blk((9, 1)),
            rep_blk((1, H)),
            rep_blk((1, 1)),
            rep_blk((1, H)),
            rep_blk((1, 1)),
        ],
        out_specs=[
            col_blk((9, BLK)),
            col_blk((1, BLK)),
            col_blk((1, BLK)),
            col_blk((N_PLACE, BLK)),
        ],
        out_shape=[
            jax.ShapeDtypeStruct((9, B), jnp.float32),
            jax.ShapeDtypeStruct((1, B), jnp.float32),
            jax.ShapeDtypeStruct((1, B), jnp.float32),
            jax.ShapeDtypeStruct((N_PLACE, B), jnp.float32),
        ],
        compiler_params=pltpu.CompilerParams(
            dimension_semantics=("parallel",)),
        interpret=interpret,
    )


def kernel(brain_state, position, place_cell_centers, place_cell_widths,
           visit_count, spatial_map, W1, b1, W2, b2, Wd, bd, Ws, bs, Wg, bg):
    pos_t = position.T
    posf = pos_t.reshape(-1)
    vcp = jnp.pad(visit_count.reshape(-1), (0, NCELL_PAD - NCELL))
    smf = spatial_map.reshape(NCELL, BRAIN_DIM)
    nov_t, mapf, parts = _sc_part_fn()(posf, vcp, smf)

    dirp_t, spd_t, gate_t, pc_t = _tc_call()(
        brain_state, pos_t,
        place_cell_centers.T,
        place_cell_widths.reshape(1, N_PLACE),
        W1, b1.reshape(1, H),
        W2, b2.reshape(1, H),
        Wd.T, bd.reshape(9, 1),
        Ws.T, bs.reshape(1, 1),
        Wg.T, bg.reshape(1, 1),
    )

    parts = parts.reshape(NC, NCELL_PAD)
    nvc = visit_count + (parts[0, :NCELL] + parts[1, :NCELL]).reshape(
        MAP_SIZE, MAP_SIZE)
    return (dirp_t.T, spd_t.T, gate_t.T, nov_t.T, pc_t.T, mapf, nvc)
